# baseline (device time: 131308 ns/iter reference)
import jax
import jax.numpy as jnp
from jax import lax
from jax.experimental import pallas as pl
from jax.experimental.pallas import tpu as pltpu

N_DEV = 8
B, SQ, D = 4, 256, 1024
H_LOC = 8
DH = 128
SCALE = 0.08838834764831843
N_CHUNK = 8
CH = (B * SQ) // N_CHUNK


def kernel(x, Wq, Wo, Wk, Wv):
    my = lax.axis_index("i")
    Wk_sl = lax.dynamic_slice_in_dim(Wk, my * 2 * DH, 2 * DH, axis=1)
    Wv_sl = lax.dynamic_slice_in_dim(Wv, my * 2 * DH, 2 * DH, axis=1)

    def body(x_ref, wq_ref, wo_ref, wk_ref, wv_ref, out_ref,
             acc_ref, comm_ref, attn_ref, send_sems, recv_sems):
        my_pos = lax.axis_index("i")
        right = (my_pos + 1) % N_DEV

        bf = jnp.bfloat16
        f32 = jnp.float32
        wq = wq_ref[...].astype(bf)
        wk = wk_ref[...].astype(bf)
        wv = wv_ref[...].astype(bf)
        wo = wo_ref[...].astype(bf)

        for b in range(B):
            xb = x_ref[b].astype(bf)
            qb = lax.dot(xb, wq, preferred_element_type=f32)
            kb = lax.dot(xb, wk, preferred_element_type=f32)
            vb = lax.dot(xb, wv, preferred_element_type=f32)
            for h in range(H_LOC):
                g = h // 4
                q = qb[:, h * DH:(h + 1) * DH].astype(bf)
                k = kb[:, g * DH:(g + 1) * DH].astype(bf)
                v = vb[:, g * DH:(g + 1) * DH].astype(bf)
                s = lax.dot_general(
                    q, k, (((1,), (1,)), ((), ())),
                    preferred_element_type=f32) * SCALE
                m = jnp.max(s, axis=-1, keepdims=True)
                p = jnp.exp(s - m)
                l = jnp.sum(p, axis=-1, keepdims=True)
                o = lax.dot(p.astype(bf), v, preferred_element_type=f32)
                attn_ref[:, h * DH:(h + 1) * DH] = o / l
            ob = attn_ref[...].astype(bf)
            pb = lax.dot(ob, wo, preferred_element_type=f32)
            acc_ref[2 * b] = pb[:CH]
            acc_ref[2 * b + 1] = pb[CH:]

        for t in range(N_DEV - 1):
            sc = (my_pos - t) % N_DEV
            rc = (my_pos - t - 1) % N_DEV
            rdma = pltpu.make_async_remote_copy(
                src_ref=acc_ref.at[sc],
                dst_ref=comm_ref.at[t],
                send_sem=send_sems.at[t],
                recv_sem=recv_sems.at[t],
                device_id=(right,),
                device_id_type=pl.DeviceIdType.MESH,
            )
            rdma.start()
            rdma.wait()
            acc_ref[rc] = acc_ref[rc] + comm_ref[t]

        for t in range(N_DEV - 1):
            sc = (my_pos + 1 - t) % N_DEV
            rdma = pltpu.make_async_remote_copy(
                src_ref=acc_ref.at[sc],
                dst_ref=acc_ref.at[sc],
                send_sem=send_sems.at[N_DEV - 1 + t],
                recv_sem=recv_sems.at[N_DEV - 1 + t],
                device_id=(right,),
                device_id_type=pl.DeviceIdType.MESH,
            )
            rdma.start()
            rdma.wait()

        for c in range(N_CHUNK):
            out_ref[c // 2, (c % 2) * CH:(c % 2) * CH + CH, :] = acc_ref[c]

    return pl.pallas_call(
        body,
        out_shape=jax.ShapeDtypeStruct((B, SQ, D), jnp.float32),
        in_specs=[pl.BlockSpec(memory_space=pltpu.VMEM)] * 5,
        out_specs=pl.BlockSpec(memory_space=pltpu.VMEM),
        scratch_shapes=[
            pltpu.VMEM((N_CHUNK, CH, D), jnp.float32),
            pltpu.VMEM((N_DEV - 1, CH, D), jnp.float32),
            pltpu.VMEM((SQ, D), jnp.float32),
            pltpu.SemaphoreType.DMA((2 * (N_DEV - 1),)),
            pltpu.SemaphoreType.DMA((2 * (N_DEV - 1),)),
        ],
    )(x, Wq, Wo, Wk_sl, Wv_sl)


# device time: 91860 ns/iter; 1.4294x vs baseline; 1.4294x over previous
import jax
import jax.numpy as jnp
from jax import lax
from jax.experimental import pallas as pl
from jax.experimental.pallas import tpu as pltpu

N_DEV = 8
B, SQ, D = 4, 256, 1024
H_LOC = 8
DH = 128
SCALE = 0.08838834764831843
N_CHUNK = 8
CH = (B * SQ) // N_CHUNK


def kernel(x, Wq, Wo, Wk, Wv):
    my = lax.axis_index("i")
    Wk_sl = lax.dynamic_slice_in_dim(Wk, my * 2 * DH, 2 * DH, axis=1)
    Wv_sl = lax.dynamic_slice_in_dim(Wv, my * 2 * DH, 2 * DH, axis=1)

    def body(x_ref, wq_ref, wo_ref, wk_ref, wv_ref, out_ref,
             acc_ref, stg_ref, comm_ref, gath_ref, attn_ref,
             send_sems, recv_sems):
        my_pos = lax.axis_index("i")
        right = (my_pos + 1) % N_DEV

        bf = jnp.bfloat16
        f32 = jnp.float32
        wq = wq_ref[...].astype(bf)
        wk = wk_ref[...].astype(bf)
        wv = wv_ref[...].astype(bf)
        wo = wo_ref[...].astype(bf)

        for b in range(B):
            xb = x_ref[b].astype(bf)
            qb = lax.dot(xb, wq, preferred_element_type=f32)
            kb = lax.dot(xb, wk, preferred_element_type=f32)
            vb = lax.dot(xb, wv, preferred_element_type=f32)
            for h in range(H_LOC):
                g = h // 4
                q = qb[:, h * DH:(h + 1) * DH].astype(bf)
                k = kb[:, g * DH:(g + 1) * DH].astype(bf)
                v = vb[:, g * DH:(g + 1) * DH].astype(bf)
                s = lax.dot_general(
                    q, k, (((1,), (1,)), ((), ())),
                    preferred_element_type=f32) * SCALE
                m = jnp.max(s, axis=-1, keepdims=True)
                p = jnp.exp(s - m)
                l = jnp.sum(p, axis=-1, keepdims=True)
                o = lax.dot(p.astype(bf), v, preferred_element_type=f32)
                attn_ref[:, h * DH:(h + 1) * DH] = o / l
            ob = attn_ref[...].astype(bf)
            pb = lax.dot(ob, wo, preferred_element_type=f32)
            acc_ref[2 * b] = pb[:CH]
            acc_ref[2 * b + 1] = pb[CH:]

        for t in range(N_DEV - 1):
            sc = (my_pos - t) % N_DEV
            rc = (my_pos - t - 1) % N_DEV
            stg_ref[t] = acc_ref[sc].astype(bf)
            rdma = pltpu.make_async_remote_copy(
                src_ref=stg_ref.at[t],
                dst_ref=comm_ref.at[t],
                send_sem=send_sems.at[t],
                recv_sem=recv_sems.at[t],
                device_id=(right,),
                device_id_type=pl.DeviceIdType.MESH,
            )
            rdma.start()
            rdma.wait()
            acc_ref[rc] = acc_ref[rc] + comm_ref[t].astype(f32)
        own = (my_pos + 1) % N_DEV
        gath_ref[own] = acc_ref[own].astype(bf)

        for t in range(N_DEV - 1):
            sc = (my_pos + 1 - t) % N_DEV
            rdma = pltpu.make_async_remote_copy(
                src_ref=gath_ref.at[sc],
                dst_ref=gath_ref.at[sc],
                send_sem=send_sems.at[N_DEV - 1 + t],
                recv_sem=recv_sems.at[N_DEV - 1 + t],
                device_id=(right,),
                device_id_type=pl.DeviceIdType.MESH,
            )
            rdma.start()
            rdma.wait()

        for c in range(N_CHUNK):
            out_ref[c // 2, (c % 2) * CH:(c % 2) * CH + CH, :] = (
                gath_ref[c].astype(f32))

    return pl.pallas_call(
        body,
        out_shape=jax.ShapeDtypeStruct((B, SQ, D), jnp.float32),
        in_specs=[pl.BlockSpec(memory_space=pltpu.VMEM)] * 5,
        out_specs=pl.BlockSpec(memory_space=pltpu.VMEM),
        scratch_shapes=[
            pltpu.VMEM((N_CHUNK, CH, D), jnp.float32),
            pltpu.VMEM((N_DEV - 1, CH, D), jnp.bfloat16),
            pltpu.VMEM((N_DEV - 1, CH, D), jnp.bfloat16),
            pltpu.VMEM((N_CHUNK, CH, D), jnp.bfloat16),
            pltpu.VMEM((SQ, D), jnp.float32),
            pltpu.SemaphoreType.DMA((2 * (N_DEV - 1),)),
            pltpu.SemaphoreType.DMA((2 * (N_DEV - 1),)),
        ],
    )(x, Wq, Wo, Wk_sl, Wv_sl)


# device time: 57587 ns/iter; 2.2802x vs baseline; 1.5952x over previous
import jax
import jax.numpy as jnp
from jax import lax
from jax.experimental import pallas as pl
from jax.experimental.pallas import tpu as pltpu

N_DEV = 8
B, SQ, D = 4, 256, 1024
H_LOC = 8
DH = 128
SCALE = 0.08838834764831843
N_CHUNK = 8
CH = (B * SQ) // N_CHUNK
HR = CH // 2

DIMS0 = (0, 1, 2)
DIMS1 = (1, 2, 0)

def _slot(c, dims):
    bits = [(c >> k) & 1 for k in range(3)]
    return (bits[dims[0]] << 2) | (bits[dims[1]] << 1) | bits[dims[2]]

SLOT0 = [_slot(c, DIMS0) for c in range(N_CHUNK)]
SLOT1 = [_slot(c, DIMS1) for c in range(N_CHUNK)]
STG_OFF = (0, 4, 6)


def kernel(x, Wq, Wo, Wk, Wv):
    my = lax.axis_index("i")
    Wk_sl = lax.dynamic_slice_in_dim(Wk, my * 2 * DH, 2 * DH, axis=1)
    Wv_sl = lax.dynamic_slice_in_dim(Wv, my * 2 * DH, 2 * DH, axis=1)

    def body(x_ref, wq_ref, wo_ref, wk_ref, wv_ref, out_ref,
             acc0, acc1, stg0, stg1, comm0, comm1, gath0, gath1, attn_ref,
             ssem0, ssem1, rsem0, rsem1):
        my_pos = lax.axis_index("i")
        d = my_pos ^ ((my_pos >> 1) & 1)

        bf = jnp.bfloat16
        f32 = jnp.float32
        wq = wq_ref[...].astype(bf)
        wk = wk_ref[...].astype(bf)
        wv = wv_ref[...].astype(bf)
        wo = wo_ref[...].astype(bf)

        for b in range(B):
            xb = x_ref[b].astype(bf)
            qb = lax.dot(xb, wq, preferred_element_type=f32)
            kb = lax.dot(xb, wk, preferred_element_type=f32)
            vb = lax.dot(xb, wv, preferred_element_type=f32)
            for h in range(H_LOC):
                g = h // 4
                q = qb[:, h * DH:(h + 1) * DH].astype(bf)
                k = kb[:, g * DH:(g + 1) * DH].astype(bf)
                v = vb[:, g * DH:(g + 1) * DH].astype(bf)
                s = lax.dot_general(
                    q, k, (((1,), (1,)), ((), ())),
                    preferred_element_type=f32) * SCALE
                m = jnp.max(s, axis=-1, keepdims=True)
                p = jnp.exp(s - m)
                l = jnp.sum(p, axis=-1, keepdims=True)
                o = lax.dot(p.astype(bf), v, preferred_element_type=f32)
                attn_ref[:, h * DH:(h + 1) * DH] = o / l
            ob = attn_ref[...].astype(bf)
            pb = lax.dot(ob, wo, preferred_element_type=f32)
            for half in range(2):
                c = 2 * b + half
                r0 = half * CH
                acc0[SLOT0[c]] = pb[r0:r0 + HR]
                acc1[SLOT1[c]] = pb[r0 + HR:r0 + CH]

        def partner(dd, dim):
            pd = dd ^ (1 << dim)
            return pd ^ ((pd >> 1) & 1)

        win0 = 0
        win1 = 0
        for j in range(3):
            half = 4 >> j
            off = STG_OFF[j]
            b0 = (d >> DIMS0[j]) & 1
            b1 = (d >> DIMS1[j]) & 1
            keep0 = win0 + half * b0
            send0 = win0 + half * (1 - b0)
            keep1 = win1 + half * b1
            send1 = win1 + half * (1 - b1)
            stg0[pl.ds(off, half)] = acc0[pl.ds(send0, half)].astype(bf)
            stg1[pl.ds(off, half)] = acc1[pl.ds(send1, half)].astype(bf)
            r0 = pltpu.make_async_remote_copy(
                src_ref=stg0.at[pl.ds(off, half)],
                dst_ref=comm0.at[pl.ds(off, half)],
                send_sem=ssem0.at[j], recv_sem=rsem0.at[j],
                device_id=(partner(d, DIMS0[j]),),
                device_id_type=pl.DeviceIdType.MESH)
            r1 = pltpu.make_async_remote_copy(
                src_ref=stg1.at[pl.ds(off, half)],
                dst_ref=comm1.at[pl.ds(off, half)],
                send_sem=ssem1.at[j], recv_sem=rsem1.at[j],
                device_id=(partner(d, DIMS1[j]),),
                device_id_type=pl.DeviceIdType.MESH)
            r0.start()
            r1.start()
            r0.wait()
            r1.wait()
            acc0[pl.ds(keep0, half)] = (
                acc0[pl.ds(keep0, half)] + comm0[pl.ds(off, half)].astype(f32))
            acc1[pl.ds(keep1, half)] = (
                acc1[pl.ds(keep1, half)] + comm1[pl.ds(off, half)].astype(f32))
            win0 = keep0
            win1 = keep1

        gath0[pl.ds(win0, 1)] = acc0[pl.ds(win0, 1)].astype(bf)
        gath1[pl.ds(win1, 1)] = acc1[pl.ds(win1, 1)].astype(bf)

        for j in range(3):
            size = 1 << j
            dim0 = DIMS0[2 - j]
            dim1 = DIMS1[2 - j]
            s0 = sum((((d >> DIMS0[i]) & 1) << (2 - i)) for i in range(3 - j))
            s1 = sum((((d >> DIMS1[i]) & 1) << (2 - i)) for i in range(3 - j))
            r0 = pltpu.make_async_remote_copy(
                src_ref=gath0.at[pl.ds(s0, size)],
                dst_ref=gath0.at[pl.ds(s0, size)],
                send_sem=ssem0.at[3 + j], recv_sem=rsem0.at[3 + j],
                device_id=(partner(d, dim0),),
                device_id_type=pl.DeviceIdType.MESH)
            r1 = pltpu.make_async_remote_copy(
                src_ref=gath1.at[pl.ds(s1, size)],
                dst_ref=gath1.at[pl.ds(s1, size)],
                send_sem=ssem1.at[3 + j], recv_sem=rsem1.at[3 + j],
                device_id=(partner(d, dim1),),
                device_id_type=pl.DeviceIdType.MESH)
            r0.start()
            r1.start()
            r0.wait()
            r1.wait()

        for c in range(N_CHUNK):
            base = (c % 2) * CH
            out_ref[c // 2, base:base + HR, :] = gath0[SLOT0[c]].astype(f32)
            out_ref[c // 2, base + HR:base + CH, :] = gath1[SLOT1[c]].astype(f32)

    return pl.pallas_call(
        body,
        out_shape=jax.ShapeDtypeStruct((B, SQ, D), jnp.float32),
        in_specs=[pl.BlockSpec(memory_space=pltpu.VMEM)] * 5,
        out_specs=pl.BlockSpec(memory_space=pltpu.VMEM),
        scratch_shapes=[
            pltpu.VMEM((N_CHUNK, HR, D), jnp.float32),
            pltpu.VMEM((N_CHUNK, HR, D), jnp.float32),
            pltpu.VMEM((7, HR, D), jnp.bfloat16),
            pltpu.VMEM((7, HR, D), jnp.bfloat16),
            pltpu.VMEM((7, HR, D), jnp.bfloat16),
            pltpu.VMEM((7, HR, D), jnp.bfloat16),
            pltpu.VMEM((N_CHUNK, HR, D), jnp.bfloat16),
            pltpu.VMEM((N_CHUNK, HR, D), jnp.bfloat16),
            pltpu.VMEM((SQ, D), jnp.float32),
            pltpu.SemaphoreType.DMA((6,)),
            pltpu.SemaphoreType.DMA((6,)),
            pltpu.SemaphoreType.DMA((6,)),
            pltpu.SemaphoreType.DMA((6,)),
        ],
    )(x, Wq, Wo, Wk_sl, Wv_sl)


# device time: 49040 ns/iter; 2.6776x vs baseline; 1.1743x over previous
import jax
import jax.numpy as jnp
from jax import lax
from jax.experimental import pallas as pl
from jax.experimental.pallas import tpu as pltpu

N_DEV = 8
B, SQ, D = 4, 256, 1024
H_LOC = 8
DH = 128
SCALE = 0.08838834764831843
N_CHUNK = 8
QR = 64
HR = 32

DIMS0 = (0, 1, 2)
DIMS1 = (1, 2, 0)


def _slot(c, dims):
    bits = [(c >> k) & 1 for k in range(3)]
    return (bits[dims[0]] << 2) | (bits[dims[1]] << 1) | bits[dims[2]]


SLOT = ([_slot(c, DIMS0) for c in range(N_CHUNK)],
        [_slot(c, DIMS1) for c in range(N_CHUNK)])
STG_OFF = (0, 4, 6)


def kernel(x, Wq, Wo, Wk, Wv):
    my = lax.axis_index("i")
    Wk_sl = lax.dynamic_slice_in_dim(Wk, my * 2 * DH, 2 * DH, axis=1)
    Wv_sl = lax.dynamic_slice_in_dim(Wv, my * 2 * DH, 2 * DH, axis=1)

    def body(x_ref, wq_ref, wo_ref, wk_ref, wv_ref, out_ref,
             acc00, acc01, acc10, acc11,
             stg00, stg01, stg10, stg11,
             comm00, comm01, comm10, comm11,
             gath00, gath01, gath10, gath11,
             attn_ref,
             ssem0, ssem1, rsem0, rsem1):
        my_pos = lax.axis_index("i")
        d = my_pos ^ ((my_pos >> 1) & 1)

        acc = ((acc00, acc01), (acc10, acc11))
        stg = ((stg00, stg01), (stg10, stg11))
        comm = ((comm00, comm01), (comm10, comm11))
        gath = ((gath00, gath01), (gath10, gath11))
        ssem = (ssem0, ssem1)
        rsem = (rsem0, rsem1)
        DIMS = (DIMS0, DIMS1)

        bf = jnp.bfloat16
        f32 = jnp.float32
        wqkv = jnp.concatenate(
            [wq_ref[...], wk_ref[...], wv_ref[...]], axis=1).astype(bf)
        wo = wo_ref[...].astype(bf)

        def bit(dim):
            return (d >> dim) & 1

        def partner(dim):
            pd = d ^ (1 << dim)
            return pd ^ ((pd >> 1) & 1)

        def compute_group(g):
            for b in (2 * g, 2 * g + 1):
                xb = x_ref[b].astype(bf)
                qkv = lax.dot(xb, wqkv, preferred_element_type=f32)
                for h in range(H_LOC):
                    gq = h // 4
                    q = qkv[:, h * DH:(h + 1) * DH].astype(bf)
                    k = qkv[:, D + gq * DH:D + (gq + 1) * DH].astype(bf)
                    v = qkv[:, D + 2 * DH + gq * DH:
                            D + 2 * DH + (gq + 1) * DH].astype(bf)
                    s = lax.dot_general(
                        q, k, (((1,), (1,)), ((), ())),
                        preferred_element_type=f32) * SCALE
                    m = jnp.max(s, axis=-1, keepdims=True)
                    p = jnp.exp(s - m)
                    l = jnp.sum(p, axis=-1, keepdims=True)
                    o = lax.dot(p.astype(bf), v, preferred_element_type=f32)
                    attn_ref[:, h * DH:(h + 1) * DH] = o / l
                ob = attn_ref[...].astype(bf)
                pb = lax.dot(ob, wo, preferred_element_type=f32)
                for qtr in range(4):
                    c = (b - 2 * g) * 4 + qtr
                    r0 = qtr * QR
                    acc[g][0][SLOT[0][c]] = pb[r0:r0 + HR]
                    acc[g][1][SLOT[1][c]] = pb[r0 + HR:r0 + QR]

        def rs_window(a, j):
            return sum((bit(DIMS[a][i]) << (2 - i)) for i in range(j))

        def rs_start(g, j):
            half = 4 >> j
            off = STG_OFF[j]
            rdmas = []
            for a in range(2):
                w = rs_window(a, j)
                send = w + half * (1 - bit(DIMS[a][j]))
                stg[g][a][pl.ds(off, half)] = (
                    acc[g][a][pl.ds(send, half)].astype(bf))
                r = pltpu.make_async_remote_copy(
                    src_ref=stg[g][a].at[pl.ds(off, half)],
                    dst_ref=comm[g][a].at[pl.ds(off, half)],
                    send_sem=ssem[g].at[6 * a + j],
                    recv_sem=rsem[g].at[6 * a + j],
                    device_id=(partner(DIMS[a][j]),),
                    device_id_type=pl.DeviceIdType.MESH)
                r.start()
                rdmas.append(r)
            return rdmas

        def rs_finish(g, j, rdmas):
            half = 4 >> j
            off = STG_OFF[j]
            for r in rdmas:
                r.wait()
            for a in range(2):
                keep = rs_window(a, j + 1)
                acc[g][a][pl.ds(keep, half)] = (
                    acc[g][a][pl.ds(keep, half)]
                    + comm[g][a][pl.ds(off, half)].astype(f32))

        def gath_seed(g):
            for a in range(2):
                own = rs_window(a, 3)
                gath[g][a][pl.ds(own, 1)] = acc[g][a][pl.ds(own, 1)].astype(bf)

        def ag_start(g, j):
            size = 1 << j
            rdmas = []
            for a in range(2):
                s = rs_window(a, 3 - j)
                r = pltpu.make_async_remote_copy(
                    src_ref=gath[g][a].at[pl.ds(s, size)],
                    dst_ref=gath[g][a].at[pl.ds(s, size)],
                    send_sem=ssem[g].at[6 * a + 3 + j],
                    recv_sem=rsem[g].at[6 * a + 3 + j],
                    device_id=(partner(DIMS[a][2 - j]),),
                    device_id_type=pl.DeviceIdType.MESH)
                r.start()
                rdmas.append(r)
            return rdmas

        def ag_finish(rdmas):
            for r in rdmas:
                r.wait()

        def write_out(g):
            for c in range(N_CHUNK):
                b = 2 * g + c // 4
                r0 = (c % 4) * QR
                out_ref[b, r0:r0 + HR, :] = gath[g][0][SLOT[0][c]].astype(f32)
                out_ref[b, r0 + HR:r0 + QR, :] = (
                    gath[g][1][SLOT[1][c]].astype(f32))

        compute_group(0)
        p0 = rs_start(0, 0)
        compute_group(1)
        p1 = rs_start(1, 0)
        rs_finish(0, 0, p0)
        p0 = rs_start(0, 1)
        rs_finish(1, 0, p1)
        p1 = rs_start(1, 1)
        rs_finish(0, 1, p0)
        p0 = rs_start(0, 2)
        rs_finish(1, 1, p1)
        p1 = rs_start(1, 2)
        rs_finish(0, 2, p0)
        gath_seed(0)
        p0 = ag_start(0, 0)
        rs_finish(1, 2, p1)
        gath_seed(1)
        p1 = ag_start(1, 0)
        ag_finish(p0)
        p0 = ag_start(0, 1)
        ag_finish(p1)
        p1 = ag_start(1, 1)
        ag_finish(p0)
        p0 = ag_start(0, 2)
        ag_finish(p1)
        p1 = ag_start(1, 2)
        ag_finish(p0)
        write_out(0)
        ag_finish(p1)
        write_out(1)

    return pl.pallas_call(
        body,
        out_shape=jax.ShapeDtypeStruct((B, SQ, D), jnp.float32),
        in_specs=[pl.BlockSpec(memory_space=pltpu.VMEM)] * 5,
        out_specs=pl.BlockSpec(memory_space=pltpu.VMEM),
        scratch_shapes=(
            [pltpu.VMEM((N_CHUNK, HR, D), jnp.float32)] * 4
            + [pltpu.VMEM((7, HR, D), jnp.bfloat16)] * 4
            + [pltpu.VMEM((7, HR, D), jnp.bfloat16)] * 4
            + [pltpu.VMEM((N_CHUNK, HR, D), jnp.bfloat16)] * 4
            + [pltpu.VMEM((SQ, D), jnp.float32)]
            + [pltpu.SemaphoreType.DMA((12,))] * 4
        ),
    )(x, Wq, Wo, Wk_sl, Wv_sl)


# device time: 48660 ns/iter; 2.6985x vs baseline; 1.0078x over previous
import jax
import jax.numpy as jnp
from jax import lax
from jax.experimental import pallas as pl
from jax.experimental.pallas import tpu as pltpu

N_DEV = 8
B, SQ, D = 4, 256, 1024
H_LOC = 8
DH = 128
SCALE = 0.08838834764831843
N_CHUNK = 8
QR = 64
HR = 32

DIMS0 = (0, 1, 2)
DIMS1 = (1, 2, 0)


def _slot(c, dims):
    bits = [(c >> k) & 1 for k in range(3)]
    return (bits[dims[0]] << 2) | (bits[dims[1]] << 1) | bits[dims[2]]


SLOT = ([_slot(c, DIMS0) for c in range(N_CHUNK)],
        [_slot(c, DIMS1) for c in range(N_CHUNK)])
STG_OFF = (0, 4, 6)


def kernel(x, Wq, Wo, Wk, Wv):
    my = lax.axis_index("i")
    Wk_sl = lax.dynamic_slice_in_dim(Wk, my * 2 * DH, 2 * DH, axis=1)
    Wv_sl = lax.dynamic_slice_in_dim(Wv, my * 2 * DH, 2 * DH, axis=1)

    def body(x_ref, wq_ref, wo_ref, wk_ref, wv_ref, out_ref,
             acc00, acc01, acc10, acc11,
             stg00, stg01, stg10, stg11,
             comm00, comm01, comm10, comm11,
             gath00, gath01, gath10, gath11,
             ssem0, ssem1, rsem0, rsem1):
        my_pos = lax.axis_index("i")
        d = my_pos ^ ((my_pos >> 1) & 1)

        acc = ((acc00, acc01), (acc10, acc11))
        stg = ((stg00, stg01), (stg10, stg11))
        comm = ((comm00, comm01), (comm10, comm11))
        gath = ((gath00, gath01), (gath10, gath11))
        ssem = (ssem0, ssem1)
        rsem = (rsem0, rsem1)
        DIMS = (DIMS0, DIMS1)

        bf = jnp.bfloat16
        f32 = jnp.float32
        wqkv = jnp.concatenate(
            [wq_ref[...], wk_ref[...], wv_ref[...]], axis=1).astype(bf)
        wo = wo_ref[...].astype(bf)

        def bit(dim):
            return (d >> dim) & 1

        def partner(dim):
            pd = d ^ (1 << dim)
            return pd ^ ((pd >> 1) & 1)

        def compute_group(g):
            for b in (2 * g, 2 * g + 1):
                xb = x_ref[b].astype(bf)
                qkv = lax.dot(xb, wqkv, preferred_element_type=f32)
                heads = []
                for h in range(H_LOC):
                    gq = h // 4
                    q = qkv[:, h * DH:(h + 1) * DH].astype(bf)
                    k = qkv[:, D + gq * DH:D + (gq + 1) * DH].astype(bf)
                    v = qkv[:, D + 2 * DH + gq * DH:
                            D + 2 * DH + (gq + 1) * DH].astype(bf)
                    s = lax.dot_general(
                        q, k, (((1,), (1,)), ((), ())),
                        preferred_element_type=f32) * SCALE
                    m = jnp.max(s, axis=-1, keepdims=True)
                    p = jnp.exp(s - m)
                    l = jnp.sum(p, axis=-1, keepdims=True)
                    o = lax.dot(p.astype(bf), v, preferred_element_type=f32)
                    heads.append((o / l).astype(bf))
                ob = jnp.concatenate(heads, axis=1)
                pb = lax.dot(ob, wo, preferred_element_type=f32)
                for qtr in range(4):
                    c = (b - 2 * g) * 4 + qtr
                    r0 = qtr * QR
                    acc[g][0][SLOT[0][c]] = pb[r0:r0 + HR]
                    acc[g][1][SLOT[1][c]] = pb[r0 + HR:r0 + QR]

        def rs_window(a, j):
            return sum((bit(DIMS[a][i]) << (2 - i)) for i in range(j))

        def rs_start(g, j):
            half = 4 >> j
            off = STG_OFF[j]
            rdmas = []
            for a in range(2):
                w = rs_window(a, j)
                send = w + half * (1 - bit(DIMS[a][j]))
                stg[g][a][pl.ds(off, half)] = (
                    acc[g][a][pl.ds(send, half)].astype(bf))
                r = pltpu.make_async_remote_copy(
                    src_ref=stg[g][a].at[pl.ds(off, half)],
                    dst_ref=comm[g][a].at[pl.ds(off, half)],
                    send_sem=ssem[g].at[6 * a + j],
                    recv_sem=rsem[g].at[6 * a + j],
                    device_id=(partner(DIMS[a][j]),),
                    device_id_type=pl.DeviceIdType.MESH)
                r.start()
                rdmas.append(r)
            return rdmas

        def rs_finish(g, j, rdmas):
            half = 4 >> j
            off = STG_OFF[j]
            for r in rdmas:
                r.wait()
            for a in range(2):
                keep = rs_window(a, j + 1)
                acc[g][a][pl.ds(keep, half)] = (
                    acc[g][a][pl.ds(keep, half)]
                    + comm[g][a][pl.ds(off, half)].astype(f32))

        def gath_seed(g):
            for a in range(2):
                own = rs_window(a, 3)
                gath[g][a][pl.ds(own, 1)] = acc[g][a][pl.ds(own, 1)].astype(bf)

        def ag_start(g, j):
            size = 1 << j
            rdmas = []
            for a in range(2):
                s = rs_window(a, 3 - j)
                r = pltpu.make_async_remote_copy(
                    src_ref=gath[g][a].at[pl.ds(s, size)],
                    dst_ref=gath[g][a].at[pl.ds(s, size)],
                    send_sem=ssem[g].at[6 * a + 3 + j],
                    recv_sem=rsem[g].at[6 * a + 3 + j],
                    device_id=(partner(DIMS[a][2 - j]),),
                    device_id_type=pl.DeviceIdType.MESH)
                r.start()
                rdmas.append(r)
            return rdmas

        def ag_finish(rdmas):
            for r in rdmas:
                r.wait()

        def write_out(g):
            for c in range(N_CHUNK):
                b = 2 * g + c // 4
                r0 = (c % 4) * QR
                out_ref[b, r0:r0 + HR, :] = gath[g][0][SLOT[0][c]]
                out_ref[b, r0 + HR:r0 + QR, :] = gath[g][1][SLOT[1][c]]

        compute_group(0)
        p0 = rs_start(0, 0)
        compute_group(1)
        p1 = rs_start(1, 0)
        rs_finish(0, 0, p0)
        p0 = rs_start(0, 1)
        rs_finish(1, 0, p1)
        p1 = rs_start(1, 1)
        rs_finish(0, 1, p0)
        p0 = rs_start(0, 2)
        rs_finish(1, 1, p1)
        p1 = rs_start(1, 2)
        rs_finish(0, 2, p0)
        gath_seed(0)
        p0 = ag_start(0, 0)
        rs_finish(1, 2, p1)
        gath_seed(1)
        p1 = ag_start(1, 0)
        ag_finish(p0)
        p0 = ag_start(0, 1)
        ag_finish(p1)
        p1 = ag_start(1, 1)
        ag_finish(p0)
        p0 = ag_start(0, 2)
        ag_finish(p1)
        p1 = ag_start(1, 2)
        ag_finish(p0)
        write_out(0)
        ag_finish(p1)
        write_out(1)

    return pl.pallas_call(
        body,
        out_shape=jax.ShapeDtypeStruct((B, SQ, D), jnp.bfloat16),
        in_specs=[pl.BlockSpec(memory_space=pltpu.VMEM)] * 5,
        out_specs=pl.BlockSpec(memory_space=pltpu.VMEM),
        scratch_shapes=(
            [pltpu.VMEM((N_CHUNK, HR, D), jnp.float32)] * 4
            + [pltpu.VMEM((7, HR, D), jnp.bfloat16)] * 4
            + [pltpu.VMEM((7, HR, D), jnp.bfloat16)] * 4
            + [pltpu.VMEM((N_CHUNK, HR, D), jnp.bfloat16)] * 4
            + [pltpu.SemaphoreType.DMA((12,))] * 4
        ),
    )(x, Wq, Wo, Wk_sl, Wv_sl)


# device time: 43683 ns/iter; 3.0059x vs baseline; 1.1139x over previous
import jax
import jax.numpy as jnp
from jax import lax
from jax.experimental import pallas as pl
from jax.experimental.pallas import tpu as pltpu

N_DEV = 8
B, SQ, D = 4, 256, 1024
H_LOC = 8
DH = 128
SCALE = 0.08838834764831843
N_CHUNK = 8
QR = 64
HR = 32

DIMS0 = (0, 1, 2)
DIMS1 = (1, 2, 0)


def _slot(c, dims):
    bits = [(c >> k) & 1 for k in range(3)]
    return (bits[dims[0]] << 2) | (bits[dims[1]] << 1) | bits[dims[2]]


SLOT = ([_slot(c, DIMS0) for c in range(N_CHUNK)],
        [_slot(c, DIMS1) for c in range(N_CHUNK)])
STG_OFF = (0, 4, 6)


def kernel(x, Wq, Wo, Wk, Wv):
    def body(x_ref, wq_ref, wo_ref, wk_ref, wv_ref, out_ref,
             acc00, acc01, acc10, acc11,
             stg00, stg01, stg10, stg11,
             comm00, comm01, comm10, comm11,
             wk_v, wv_v, wo_v,
             ssem0, ssem1, rsem0, rsem1, csem):
        my_pos = lax.axis_index("i")
        d = my_pos ^ ((my_pos >> 1) & 1)

        acc = ((acc00, acc01), (acc10, acc11))
        stg = ((stg00, stg01), (stg10, stg11))
        comm = ((comm00, comm01), (comm10, comm11))
        ssem = (ssem0, ssem1)
        rsem = (rsem0, rsem1)
        DIMS = (DIMS0, DIMS1)

        bf = jnp.bfloat16
        f32 = jnp.float32

        def bit(dim):
            return (d >> dim) & 1

        def partner(dim):
            pd = d ^ (1 << dim)
            return pd ^ ((pd >> 1) & 1)

        bsem = pltpu.get_barrier_semaphore()
        for dim in range(3):
            pl.semaphore_signal(
                bsem, inc=1, device_id=(partner(dim),),
                device_id_type=pl.DeviceIdType.MESH)

        cps = [
            pltpu.make_async_copy(
                wk_ref.at[:, pl.ds(my_pos * 2 * DH, 2 * DH)], wk_v,
                csem.at[0]),
            pltpu.make_async_copy(
                wv_ref.at[:, pl.ds(my_pos * 2 * DH, 2 * DH)], wv_v,
                csem.at[1]),
            pltpu.make_async_copy(wo_ref, wo_v, csem.at[2]),
        ]
        for cp in cps:
            cp.start()

        wq = (wq_ref[...] * SCALE).astype(bf)

        wcache = {}

        def kv_weights():
            if not wcache:
                for cp in cps:
                    cp.wait()
                wcache["wk"] = wk_v[...].astype(bf)
                wcache["wv"] = wv_v[...].astype(bf)
                wcache["wo"] = wo_v[...].astype(bf)
            return wcache

        def compute_group(g):
            for b in (2 * g, 2 * g + 1):
                xb = x_ref[b].astype(bf)
                qb = lax.dot(xb, wq, preferred_element_type=f32).astype(bf)
                w = kv_weights()
                kb = lax.dot(xb, w["wk"], preferred_element_type=f32).astype(bf)
                vb = lax.dot(xb, w["wv"], preferred_element_type=f32).astype(bf)
                heads = []
                for gq in range(2):
                    q4 = jnp.concatenate(
                        [qb[:, (4 * gq + i) * DH:(4 * gq + i + 1) * DH]
                         for i in range(4)], axis=0)
                    k = kb[:, gq * DH:(gq + 1) * DH]
                    v = vb[:, gq * DH:(gq + 1) * DH]
                    s = lax.dot_general(
                        q4, k, (((1,), (1,)), ((), ())),
                        preferred_element_type=f32)
                    m = jnp.max(s, axis=-1, keepdims=True)
                    p = jnp.exp((s - m).astype(bf))
                    l = jnp.sum(p.astype(f32), axis=-1, keepdims=True)
                    o = lax.dot(p, v, preferred_element_type=f32)
                    o = (o * (1.0 / l)).astype(bf)
                    heads.extend(
                        o[i * SQ:(i + 1) * SQ] for i in range(4))
                ob = jnp.concatenate(heads, axis=1)
                pb = lax.dot(ob, w["wo"],
                             preferred_element_type=f32)
                for qtr in range(4):
                    c = (b - 2 * g) * 4 + qtr
                    r0 = qtr * QR
                    acc[g][0][SLOT[0][c]] = pb[r0:r0 + HR]
                    acc[g][1][SLOT[1][c]] = pb[r0 + HR:r0 + QR]

        def rs_window(a, j):
            return sum((bit(DIMS[a][i]) << (2 - i)) for i in range(j))

        def rs_issue(g, a, j):
            half = 4 >> j
            off = STG_OFF[j]
            send = rs_window(a, j) + half * (1 - bit(DIMS[a][j]))
            stg[g][a][pl.ds(off, half)] = (
                acc[g][a][pl.ds(send, half)].astype(bf))
            r = pltpu.make_async_remote_copy(
                src_ref=stg[g][a].at[pl.ds(off, half)],
                dst_ref=comm[g][a].at[pl.ds(off, half)],
                send_sem=ssem[g].at[6 * a + j],
                recv_sem=rsem[g].at[6 * a + j],
                device_id=(partner(DIMS[a][j]),),
                device_id_type=pl.DeviceIdType.MESH)
            r.start()
            return r

        def chunk_of_slot(a, s):
            if a == 0:
                return ((s >> 2) & 1) | (s & 2) | ((s & 1) << 2)
            return (s & 1) | (((s >> 2) & 1) << 1) | (((s >> 1) & 1) << 2)

        def out_region(g, a, s):
            c = chunk_of_slot(a, s)
            bb = 2 * g + (c >> 2)
            rr = (c & 3) * QR + a * HR
            return bb, rr

        def ag_issue(g, a, j):
            size = 1 << j
            w = rs_window(a, 3 - j)
            rdmas = []
            for i in range(size):
                bb, rr = out_region(g, a, w + i)
                r = pltpu.make_async_remote_copy(
                    src_ref=out_ref.at[bb, pl.ds(rr, HR)],
                    dst_ref=out_ref.at[bb, pl.ds(rr, HR)],
                    send_sem=ssem[g].at[6 * a + 3 + j],
                    recv_sem=rsem[g].at[6 * a + 3 + j],
                    device_id=(partner(DIMS[a][2 - j]),),
                    device_id_type=pl.DeviceIdType.MESH)
                r.start()
                rdmas.append(r)
            return rdmas

        def rs_start(g):
            return [rs_issue(g, a, 0) for a in range(2)]

        def rs_step(g, j, rdmas):
            half = 4 >> j
            off = STG_OFF[j]
            nxt = []
            for a in range(2):
                rdmas[a].wait()
                keep = rs_window(a, j + 1)
                acc[g][a][pl.ds(keep, half)] = (
                    acc[g][a][pl.ds(keep, half)]
                    + comm[g][a][pl.ds(off, half)].astype(f32))
                if j < 2:
                    nxt.append(rs_issue(g, a, j + 1))
                else:
                    own = rs_window(a, 3)
                    bb, rr = out_region(g, a, own)
                    out_ref[bb, pl.ds(rr, HR), :] = (
                        acc[g][a][own].astype(bf))
                    nxt.append(ag_issue(g, a, 0))
            return nxt

        def ag_step(g, j, rdmas):
            nxt = []
            for a in range(2):
                for r in rdmas[a]:
                    r.wait()
                if j < 2:
                    nxt.append(ag_issue(g, a, j + 1))
            return nxt

        compute_group(0)
        pl.semaphore_wait(bsem, 3)
        p0 = rs_start(0)
        compute_group(1)
        p1 = rs_start(1)
        p0 = rs_step(0, 0, p0)
        p1 = rs_step(1, 0, p1)
        p0 = rs_step(0, 1, p0)
        p1 = rs_step(1, 1, p1)
        p0 = rs_step(0, 2, p0)
        p1 = rs_step(1, 2, p1)
        p0 = ag_step(0, 0, p0)
        p1 = ag_step(1, 0, p1)
        p0 = ag_step(0, 1, p0)
        p1 = ag_step(1, 1, p1)
        ag_step(0, 2, p0)
        ag_step(1, 2, p1)

    return pl.pallas_call(
        body,
        out_shape=jax.ShapeDtypeStruct((B, SQ, D), jnp.bfloat16),
        in_specs=(
            [pl.BlockSpec(memory_space=pltpu.VMEM)] * 2
            + [pl.BlockSpec(memory_space=pl.ANY)] * 3
        ),
        out_specs=pl.BlockSpec(memory_space=pltpu.VMEM),
        scratch_shapes=(
            [pltpu.VMEM((N_CHUNK, HR, D), jnp.float32)] * 4
            + [pltpu.VMEM((7, HR, D), jnp.bfloat16)] * 4
            + [pltpu.VMEM((7, HR, D), jnp.bfloat16)] * 4
            + [pltpu.VMEM((D, 2 * DH), jnp.float32)] * 2
            + [pltpu.VMEM((D, D), jnp.float32)]
            + [pltpu.SemaphoreType.DMA((12,))] * 4
            + [pltpu.SemaphoreType.DMA((3,))]
        ),
        compiler_params=pltpu.CompilerParams(collective_id=0),
    )(x, Wq, Wo, Wk, Wv)


# device time: 41391 ns/iter; 3.1724x vs baseline; 1.0554x over previous
import jax
import jax.numpy as jnp
from jax import lax
from jax.experimental import pallas as pl
from jax.experimental.pallas import tpu as pltpu

N_DEV = 8
B, SQ, D = 4, 256, 1024
H_LOC = 8
DH = 128
SCALE = 0.08838834764831843
N_CHUNK = 8
NG = 4
QR = 32
HR = 16

DIMS0 = (0, 1, 2)
DIMS1 = (1, 2, 0)


def _slot(c, dims):
    bits = [(c >> k) & 1 for k in range(3)]
    return (bits[dims[0]] << 2) | (bits[dims[1]] << 1) | bits[dims[2]]


SLOT = ([_slot(c, DIMS0) for c in range(N_CHUNK)],
        [_slot(c, DIMS1) for c in range(N_CHUNK)])
STG_OFF = (0, 4, 6)


def kernel(x, Wq, Wo, Wk, Wv):
    def body(x_ref, wq_ref, wo_ref, wk_ref, wv_ref, out_ref,
             acc00, acc01, acc10, acc11, acc20, acc21, acc30, acc31,
             comm00, comm01, comm10, comm11,
             comm20, comm21, comm30, comm31,
             wk_v, wv_v, wo_v,
             ssem0, ssem1, ssem2, ssem3,
             rsem0, rsem1, rsem2, rsem3, csem):
        my_pos = lax.axis_index("i")
        d = my_pos ^ ((my_pos >> 1) & 1)

        acc = ((acc00, acc01), (acc10, acc11), (acc20, acc21), (acc30, acc31))
        comm = ((comm00, comm01), (comm10, comm11),
                (comm20, comm21), (comm30, comm31))
        ssem = (ssem0, ssem1, ssem2, ssem3)
        rsem = (rsem0, rsem1, rsem2, rsem3)
        DIMS = (DIMS0, DIMS1)

        bf = jnp.bfloat16
        f32 = jnp.float32

        def bit(dim):
            return (d >> dim) & 1

        def partner(dim):
            pd = d ^ (1 << dim)
            return pd ^ ((pd >> 1) & 1)

        bsem = pltpu.get_barrier_semaphore()
        for dim in range(3):
            pl.semaphore_signal(
                bsem, inc=1, device_id=(partner(dim),),
                device_id_type=pl.DeviceIdType.MESH)

        cps = [
            pltpu.make_async_copy(
                wk_ref.at[:, pl.ds(my_pos * 2 * DH, 2 * DH)], wk_v,
                csem.at[0]),
            pltpu.make_async_copy(
                wv_ref.at[:, pl.ds(my_pos * 2 * DH, 2 * DH)], wv_v,
                csem.at[1]),
            pltpu.make_async_copy(wo_ref, wo_v, csem.at[2]),
        ]
        for cp in cps:
            cp.start()

        wq = (wq_ref[...] * SCALE).astype(bf)

        wcache = {}

        def kv_weights():
            if not wcache:
                for cp in cps:
                    cp.wait()
                wcache["wk"] = wk_v[...].astype(bf)
                wcache["wv"] = wv_v[...].astype(bf)
                wcache["wo"] = wo_v[...].astype(bf)
            return wcache

        def compute_group(g):
            b = g
            xb = x_ref[b].astype(bf)
            qb = lax.dot(xb, wq, preferred_element_type=f32).astype(bf)
            w = kv_weights()
            kb = lax.dot(xb, w["wk"], preferred_element_type=f32).astype(bf)
            vb = lax.dot(xb, w["wv"], preferred_element_type=f32).astype(bf)
            heads = []
            for gq in range(2):
                q4 = jnp.concatenate(
                    [qb[:, (4 * gq + i) * DH:(4 * gq + i + 1) * DH]
                     for i in range(4)], axis=0)
                k = kb[:, gq * DH:(gq + 1) * DH]
                v = vb[:, gq * DH:(gq + 1) * DH]
                s = lax.dot_general(
                    q4, k, (((1,), (1,)), ((), ())),
                    preferred_element_type=f32)
                m = jnp.max(s, axis=-1, keepdims=True)
                p = jnp.exp((s - m).astype(bf))
                l = jnp.sum(p.astype(f32), axis=-1, keepdims=True)
                o = lax.dot(p, v, preferred_element_type=f32)
                o = (o * (1.0 / l)).astype(bf)
                heads.extend(
                    o[i * SQ:(i + 1) * SQ] for i in range(4))
            ob = jnp.concatenate(heads, axis=1)
            pb = lax.dot(ob, w["wo"],
                         preferred_element_type=f32)
            pbb = pb.astype(bf)
            for c in range(N_CHUNK):
                r0 = c * QR
                acc[g][0][SLOT[0][c]] = pbb[r0:r0 + HR]
                acc[g][1][SLOT[1][c]] = pbb[r0 + HR:r0 + QR]

        def rs_window(a, j):
            return sum((bit(DIMS[a][i]) << (2 - i)) for i in range(j))

        def rs_issue(g, a, j):
            half = 4 >> j
            off = STG_OFF[j]
            send = rs_window(a, j) + half * (1 - bit(DIMS[a][j]))
            r = pltpu.make_async_remote_copy(
                src_ref=acc[g][a].at[pl.ds(send, half)],
                dst_ref=comm[g][a].at[pl.ds(off, half)],
                send_sem=ssem[g].at[6 * a + j],
                recv_sem=rsem[g].at[6 * a + j],
                device_id=(partner(DIMS[a][j]),),
                device_id_type=pl.DeviceIdType.MESH)
            r.start()
            return r

        def chunk_of_slot(a, s):
            if a == 0:
                return ((s >> 2) & 1) | (s & 2) | ((s & 1) << 2)
            return (s & 1) | (((s >> 2) & 1) << 1) | (((s >> 1) & 1) << 2)

        def out_region(g, a, s):
            c = chunk_of_slot(a, s)
            rr = c * QR + a * HR
            return g, rr

        def ag_issue(g, a, j):
            size = 1 << j
            w = rs_window(a, 3 - j)
            rdmas = []
            for i in range(size):
                bb, rr = out_region(g, a, w + i)
                r = pltpu.make_async_remote_copy(
                    src_ref=out_ref.at[bb, pl.ds(rr, HR)],
                    dst_ref=out_ref.at[bb, pl.ds(rr, HR)],
                    send_sem=ssem[g].at[6 * a + 3 + j],
                    recv_sem=rsem[g].at[6 * a + 3 + j],
                    device_id=(partner(DIMS[a][2 - j]),),
                    device_id_type=pl.DeviceIdType.MESH)
                r.start()
                rdmas.append(r)
            return rdmas

        def rs_start(g):
            return [rs_issue(g, a, 0) for a in range(2)]

        def rs_step(g, j, rdmas):
            half = 4 >> j
            off = STG_OFF[j]
            nxt = []
            for a in range(2):
                rdmas[a].wait()
                keep = rs_window(a, j + 1)
                acc[g][a][pl.ds(keep, half)] = (
                    acc[g][a][pl.ds(keep, half)]
                    + comm[g][a][pl.ds(off, half)])
                if j < 2:
                    nxt.append(rs_issue(g, a, j + 1))
                else:
                    own = rs_window(a, 3)
                    bb, rr = out_region(g, a, own)
                    out_ref[bb, pl.ds(rr, HR), :] = acc[g][a][own]
                    nxt.append(ag_issue(g, a, 0))
            return nxt

        def ag_step(g, j, rdmas):
            nxt = []
            for a in range(2):
                for r in rdmas[a]:
                    r.wait()
                if j < 2:
                    nxt.append(ag_issue(g, a, j + 1))
            return nxt

        compute_group(0)
        pl.semaphore_wait(bsem, 3)
        p = [None] * NG
        p[0] = rs_start(0)
        for g in range(1, NG):
            compute_group(g)
            p[g] = rs_start(g)
        for j in range(3):
            for g in range(NG):
                p[g] = rs_step(g, j, p[g])
        for j in range(3):
            for g in range(NG):
                p[g] = ag_step(g, j, p[g])

    return pl.pallas_call(
        body,
        out_shape=jax.ShapeDtypeStruct((B, SQ, D), jnp.bfloat16),
        in_specs=(
            [pl.BlockSpec(memory_space=pltpu.VMEM)] * 2
            + [pl.BlockSpec(memory_space=pl.ANY)] * 3
        ),
        out_specs=pl.BlockSpec(memory_space=pltpu.VMEM),
        scratch_shapes=(
            [pltpu.VMEM((N_CHUNK, HR, D), jnp.bfloat16)] * 8
            + [pltpu.VMEM((7, HR, D), jnp.bfloat16)] * 8
            + [pltpu.VMEM((D, 2 * DH), jnp.float32)] * 2
            + [pltpu.VMEM((D, D), jnp.float32)]
            + [pltpu.SemaphoreType.DMA((12,))] * 8
            + [pltpu.SemaphoreType.DMA((3,))]
        ),
        compiler_params=pltpu.CompilerParams(collective_id=0),
    )(x, Wq, Wo, Wk, Wv)
